# Initial kernel scaffold; baseline (speedup 1.0000x reference)
import jax
import jax.numpy as jnp
from jax.experimental import pallas as pl


def kernel(entities, adjacencies, W_enc0, b_enc0, W_dec0, b_dec0, W_enc1, b_enc1, W_dec1, b_dec1, W_enc2, b_enc2, W_dec2, b_dec2, W_proj, b_proj, W_out, b_out):
    raise NotImplementedError("write your pallas kernel here")


# speedup vs baseline: 8.0100x; 8.0100x over previous
import functools

import jax
import jax.numpy as jnp
from jax import lax
from jax.experimental import pallas as pl
from jax.experimental.pallas import tpu as pltpu
from jax.experimental.pallas import tpu_sc as plsc

N = 10000
E = 320000
D = 128
BN = 64
IN1 = D + 2 * BN
IN2 = IN1 + 2 * BN
P = D * 3

NC = 2
NS = 16
EPW = E // NS
CH = 80
NCH = EPW // CH
NP = 10240
RPS = NP // NS
CW = 16

BLK = 2000


def _relu(x):
    return jnp.maximum(x, 0.0)


def _dot(a, b):
    return jnp.dot(a, b, preferred_element_type=jnp.float32)


_MESH = plsc.VectorSubcoreMesh(core_axis_name="c", subcore_axis_name="s")
_SC_PARAMS = pltpu.CompilerParams(use_tc_tiling_on_sc=False)


def _make_seg(with_counts):
    out_type = [jax.ShapeDtypeStruct((NC, NP, BN), jnp.float32)]
    scratch = [
        pltpu.VMEM((NCH, CH), jnp.int32),
        pltpu.VMEM((NCH, CH), jnp.int32),
        pltpu.VMEM((CH, BN), jnp.float32),
        pltpu.SemaphoreType.DMA,
        pltpu.VMEM_SHARED((NP, BN), jnp.float32),
    ]
    if with_counts:
        out_type.append(jax.ShapeDtypeStruct((NC, NP, CW), jnp.float32))
        scratch += [
            pltpu.VMEM((CH, CW), jnp.float32),
            pltpu.VMEM_SHARED((NP, CW), jnp.float32),
        ]

    @functools.partial(pl.kernel, mesh=_MESH, out_type=out_type,
                       scratch_types=scratch, compiler_params=_SC_PARAMS)
    def seg(*refs):
        if with_counts:
            (src_r, dst_r, z, zeros, zeros_c, ones,
             agg, cnt, idx_g, idx_sc, rows, sem, acc, ones_v, acc_c) = refs
        else:
            (src_r, dst_r, z, zeros,
             agg, idx_g, idx_sc, rows, sem, acc) = refs
        c = lax.axis_index("c")
        s = lax.axis_index("s")
        r0 = s * RPS
        pltpu.sync_copy(zeros.at[pl.ds(r0, RPS)], acc.at[pl.ds(r0, RPS)])
        if with_counts:
            pltpu.sync_copy(zeros_c.at[pl.ds(r0, RPS)],
                            acc_c.at[pl.ds(r0, RPS)])
            pltpu.sync_copy(ones, ones_v)

        @pl.when(c == 0)
        def _():
            pltpu.sync_copy(src_r.at[s], idx_g)
            pltpu.sync_copy(dst_r.at[s], idx_sc)

        @pl.when(c != 0)
        def _():
            pltpu.sync_copy(dst_r.at[s], idx_g)
            pltpu.sync_copy(src_r.at[s], idx_sc)

        plsc.subcore_barrier()

        if with_counts:
            def body(j, carry):
                pltpu.async_copy(z.at[idx_g.at[j]], rows, sem).wait()
                pltpu.sync_copy(rows, acc.at[idx_sc.at[j]], add=True)
                pltpu.sync_copy(ones_v, acc_c.at[idx_sc.at[j]], add=True)
                return carry
        else:
            def body(j, carry):
                pltpu.async_copy(z.at[idx_g.at[j]], rows, sem).wait()
                pltpu.sync_copy(rows, acc.at[idx_sc.at[j]], add=True)
                return carry

        lax.fori_loop(0, NCH, body, 0)
        plsc.subcore_barrier()
        pltpu.sync_copy(acc.at[pl.ds(r0, RPS)], agg.at[c, pl.ds(r0, RPS)])
        if with_counts:
            pltpu.sync_copy(acc_c.at[pl.ds(r0, RPS)],
                            cnt.at[c, pl.ds(r0, RPS)])

    return seg


_seg_with_counts = _make_seg(True)
_seg_no_counts = _make_seg(False)


def _means(agg, cnt):
    stgt = agg[0] / jnp.maximum(cnt[0, :, 0:1], 1.0)
    ssrc = agg[1] / jnp.maximum(cnt[1, :, 0:1], 1.0)
    return ssrc, stgt


def _enc0_body(x, we, be, wd, bd, z_ref, h_ref):
    z = _relu(_dot(x[...], we[...]) + be[...])
    z_ref[...] = z
    h_ref[...] = _relu(_dot(z, wd[...]) + bd[...])


def _enc0(x, we, be, wd, bd):
    return pl.pallas_call(
        _enc0_body,
        grid=(N // BLK,),
        in_specs=[
            pl.BlockSpec((BLK, D), lambda i: (i, 0)),
            pl.BlockSpec((D, BN), lambda i: (0, 0)),
            pl.BlockSpec((1, BN), lambda i: (0, 0)),
            pl.BlockSpec((BN, D), lambda i: (0, 0)),
            pl.BlockSpec((1, D), lambda i: (0, 0)),
        ],
        out_specs=[
            pl.BlockSpec((BLK, BN), lambda i: (i, 0)),
            pl.BlockSpec((BLK, D), lambda i: (i, 0)),
        ],
        out_shape=[
            jax.ShapeDtypeStruct((N, BN), jnp.float32),
            jax.ShapeDtypeStruct((N, D), jnp.float32),
        ],
    )(x, we, be, wd, bd)


def _depth1_body(h0, agg, cnt, weh, wes, wet, be, wd, bd, z_ref, h_ref):
    ssrc, stgt = _means(agg[...], cnt[...])
    z = _relu(_dot(h0[...], weh[...]) + _dot(ssrc, wes[...])
              + _dot(stgt, wet[...]) + be[...])
    z_ref[...] = z
    h_ref[...] = _relu(_dot(z, wd[...]) + bd[...])


def _depth1(h0, agg, cnt, weh, wes, wet, be, wd, bd):
    return pl.pallas_call(
        _depth1_body,
        grid=(N // BLK,),
        in_specs=[
            pl.BlockSpec((BLK, D), lambda i: (i, 0)),
            pl.BlockSpec((NC, BLK, BN), lambda i: (0, i, 0)),
            pl.BlockSpec((NC, BLK, CW), lambda i: (0, i, 0)),
            pl.BlockSpec((D, BN), lambda i: (0, 0)),
            pl.BlockSpec((BN, BN), lambda i: (0, 0)),
            pl.BlockSpec((BN, BN), lambda i: (0, 0)),
            pl.BlockSpec((1, BN), lambda i: (0, 0)),
            pl.BlockSpec((BN, IN1), lambda i: (0, 0)),
            pl.BlockSpec((1, IN1), lambda i: (0, 0)),
        ],
        out_specs=[
            pl.BlockSpec((BLK, BN), lambda i: (i, 0)),
            pl.BlockSpec((BLK, IN1), lambda i: (i, 0)),
        ],
        out_shape=[
            jax.ShapeDtypeStruct((N, BN), jnp.float32),
            jax.ShapeDtypeStruct((N, IN1), jnp.float32),
        ],
    )(h0, agg, cnt, weh, wes, wet, be, wd, bd)


def _final_body(h0, h1, agg, cnt, weh, wes, wet, be, wd, bd,
                wp0, wp1, wp2, bp, wo, bo, out_ref):
    ssrc, stgt = _means(agg[...], cnt[...])
    z2 = _relu(_dot(h1[...], weh[...]) + _dot(ssrc, wes[...])
               + _dot(stgt, wet[...]) + be[...])
    h2 = _relu(_dot(z2, wd[...]) + bd[...])
    proj = _relu(_dot(h0[...], wp0[...]) + _dot(h1[...], wp1[...])
                 + _dot(h2, wp2[...]) + bp[...])
    out_ref[...] = _dot(proj, wo[...]) + bo[...]


def _final(h0, h1, agg, cnt, weh, wes, wet, be, wd, bd,
           wp0, wp1, wp2, bp, wo, bo):
    return pl.pallas_call(
        _final_body,
        grid=(N // BLK,),
        in_specs=[
            pl.BlockSpec((BLK, D), lambda i: (i, 0)),
            pl.BlockSpec((BLK, IN1), lambda i: (i, 0)),
            pl.BlockSpec((NC, BLK, BN), lambda i: (0, i, 0)),
            pl.BlockSpec((NC, BLK, CW), lambda i: (0, i, 0)),
            pl.BlockSpec((IN1, BN), lambda i: (0, 0)),
            pl.BlockSpec((BN, BN), lambda i: (0, 0)),
            pl.BlockSpec((BN, BN), lambda i: (0, 0)),
            pl.BlockSpec((1, BN), lambda i: (0, 0)),
            pl.BlockSpec((BN, IN2), lambda i: (0, 0)),
            pl.BlockSpec((1, IN2), lambda i: (0, 0)),
            pl.BlockSpec((D, P), lambda i: (0, 0)),
            pl.BlockSpec((IN1, P), lambda i: (0, 0)),
            pl.BlockSpec((IN2, P), lambda i: (0, 0)),
            pl.BlockSpec((1, P), lambda i: (0, 0)),
            pl.BlockSpec((P, D), lambda i: (0, 0)),
            pl.BlockSpec((1, D), lambda i: (0, 0)),
        ],
        out_specs=pl.BlockSpec((BLK, D), lambda i: (i, 0)),
        out_shape=jax.ShapeDtypeStruct((N, D), jnp.float32),
    )(h0, h1, agg, cnt, weh, wes, wet, be, wd, bd,
      wp0, wp1, wp2, bp, wo, bo)


def kernel(entities, adjacencies, W_enc0, b_enc0, W_dec0, b_dec0,
           W_enc1, b_enc1, W_dec1, b_dec1, W_enc2, b_enc2, W_dec2, b_dec2,
           W_proj, b_proj, W_out, b_out):
    src_r = adjacencies[0].reshape(NS, NCH, CH)
    dst_r = adjacencies[1].reshape(NS, NCH, CH)
    zeros = jnp.zeros((NP, BN), jnp.float32)
    zeros_c = jnp.zeros((NP, CW), jnp.float32)
    ones = jnp.ones((CH, CW), jnp.float32)

    z0, h0 = _enc0(entities, W_enc0, b_enc0.reshape(1, -1),
                   W_dec0, b_dec0.reshape(1, -1))
    agg0, cnt = _seg_with_counts(src_r, dst_r, z0, zeros, zeros_c, ones)
    z1, h1 = _depth1(h0, agg0, cnt,
                     W_enc1[:D], W_enc1[D:D + BN], W_enc1[D + BN:],
                     b_enc1.reshape(1, -1), W_dec1, b_dec1.reshape(1, -1))
    agg1 = _seg_no_counts(src_r, dst_r, z1, zeros)[0]
    recon = _final(h0, h1, agg1, cnt,
                   W_enc2[:IN1], W_enc2[IN1:IN1 + BN], W_enc2[IN1 + BN:],
                   b_enc2.reshape(1, -1), W_dec2, b_dec2.reshape(1, -1),
                   W_proj[:D], W_proj[D:D + IN1], W_proj[D + IN1:],
                   b_proj.reshape(1, -1), W_out, b_out.reshape(1, -1))
    return recon
